# baseline (speedup 1.0000x reference)
import functools

import jax
import jax.numpy as jnp
from jax import lax
from jax.experimental import pallas as pl
from jax.experimental.pallas import tpu as pltpu
from jax.experimental.pallas import tpu_sc as plsc

N_NODES = 10000
N_EDGES = 320000
D = 128
NPAD = 10240

NC = 2
NS = 16
NW = NC * NS
EPW = N_EDGES // NW
CHUNK = 80
NCHD = EPW // CHUNK
RING = 3
PREF = 2
ROWS_PER_TILE = NPAD // NS

EBLK = 8000
NBLK = 1000


def _msg_body(et_ref, h_ref, s_ref, d_ref, weoh_ref, beoh_ref, wel_ref,
              bel_ref, out_ref):
    et = et_ref[0, 0, :][:, None]
    w0 = weoh_ref[0:1, :]
    w1 = weoh_ref[1:2, :]
    w2 = weoh_ref[2:3, :]
    etw = jnp.where(et == 0, w0, jnp.where(et == 1, w1, w2))
    a = s_ref[...] * d_ref[...] + etw + beoh_ref[...]
    ew = jnp.dot(a.astype(jnp.bfloat16), wel_ref[...].astype(jnp.bfloat16),
                 preferred_element_type=jnp.float32)
    out_ref[...] = h_ref[...] * (ew + bel_ref[...])


def _msg_stage(h_child, src_node, dst_node, edge_type, W_eoh, bW_eoh, W_el,
               bW_el):
    n_blk = N_EDGES // EBLK
    et3 = edge_type.reshape(n_blk, 1, EBLK)
    weoh_p = jnp.zeros((8, D), jnp.float32).at[:3].set(W_eoh)
    return pl.pallas_call(
        _msg_body,
        grid=(n_blk,),
        in_specs=[
            pl.BlockSpec((1, 1, EBLK), lambda i: (i, 0, 0)),
            pl.BlockSpec((EBLK, D), lambda i: (i, 0)),
            pl.BlockSpec((EBLK, D), lambda i: (i, 0)),
            pl.BlockSpec((EBLK, D), lambda i: (i, 0)),
            pl.BlockSpec((8, D), lambda i: (0, 0)),
            pl.BlockSpec((1, D), lambda i: (0, 0)),
            pl.BlockSpec((D, D), lambda i: (0, 0)),
            pl.BlockSpec((1, D), lambda i: (0, 0)),
        ],
        out_specs=pl.BlockSpec((EBLK, D), lambda i: (i, 0)),
        out_shape=jax.ShapeDtypeStruct((N_EDGES, D), jnp.float32),
    )(et3, h_child, src_node, dst_node, weoh_p, bW_eoh.reshape(1, D), W_el,
      bW_el.reshape(1, D))


def _sc_segsum(arrays, ids2d, zeros_hbm):
    n_arr = len(arrays)
    mesh = plsc.VectorSubcoreMesh(
        core_axis_name="c", subcore_axis_name="s", num_cores=NC,
        num_subcores=NS)

    slab_types = [pltpu.VMEM((CHUNK, D), jnp.float32)] * RING
    sem_types = [pltpu.SemaphoreType.DMA] * (2 * RING)

    @functools.partial(
        pl.kernel,
        out_type=jax.ShapeDtypeStruct((n_arr, NC, NPAD, D), jnp.float32),
        mesh=mesh,
        scratch_types=[
            pltpu.VMEM_SHARED((NPAD, D), jnp.float32),
            pltpu.VMEM((NCHD, CHUNK), jnp.int32),
        ] + slab_types + sem_types,
    )
    def k(*refs):
        val_hbms = refs[:n_arr]
        ids_hbm, zero_hbm, out_hbm, acc, ids_v = refs[n_arr:n_arr + 5]
        slab = refs[n_arr + 5:n_arr + 5 + RING]
        sem_ld = refs[n_arr + 5 + RING:n_arr + 5 + 2 * RING]
        sem_sc = refs[n_arr + 5 + 2 * RING:n_arr + 5 + 3 * RING]
        cid = lax.axis_index("c")
        sid = lax.axis_index("s")
        wid = sid * NC + cid
        base = wid * EPW
        row0 = sid * ROWS_PER_TILE

        pltpu.sync_copy(ids_hbm.at[wid], ids_v)

        def zero_acc():
            pltpu.sync_copy(zero_hbm.at[pl.ds(row0, ROWS_PER_TILE)],
                            acc.at[pl.ds(row0, ROWS_PER_TILE)])

        def load(vals, t, b):
            pltpu.async_copy(vals.at[pl.ds(base + t * CHUNK, CHUNK)],
                             slab[b], sem_ld[b])

        def wait_load(vals, b):
            pltpu.make_async_copy(vals.at[pl.ds(base, CHUNK)], slab[b],
                                  sem_ld[b]).wait()

        def fire_scatter(c, b):
            pltpu.async_copy(slab[b], acc.at[ids_v.at[c]], sem_sc[b],
                             add=True)

        def wait_scatter(b):
            pltpu.make_async_copy(slab[b], acc.at[ids_v.at[0]],
                                  sem_sc[b]).wait()

        zero_acc()
        plsc.subcore_barrier()

        for j, vals in enumerate(val_hbms):
            for i in range(PREF):
                load(vals, i, i)

            def grp(g, _, vals=vals):
                for i in range(RING):
                    c = RING * g + i
                    wait_load(vals, i)
                    fire_scatter(c, i)
                    t = c + PREF
                    bt = (i + PREF) % RING

                    def ws_ld(t=t, bt=bt):
                        wait_scatter(bt)
                        load(vals, t, bt)

                    if i == 0:
                        @pl.when(g > 0)
                        def _():
                            ws_ld()

                        @pl.when(g == 0)
                        def _(t=t, bt=bt):
                            load(vals, t, bt)
                    else:
                        ws_ld()
                return 0

            lax.fori_loop(0, NCHD // RING, grp, 0)
            for c in range(RING * (NCHD // RING), NCHD):
                b = c % RING
                wait_load(vals, b)
                fire_scatter(c, b)
            for b in range(RING):
                wait_scatter(b)
            plsc.subcore_barrier()
            pltpu.sync_copy(
                acc.at[pl.ds(row0, ROWS_PER_TILE)],
                out_hbm.at[j, cid, pl.ds(row0, ROWS_PER_TILE)])
            if j < n_arr - 1:
                zero_acc()
                plsc.subcore_barrier()

    return k(*arrays, ids2d, zeros_hbm)


def _gates_body(s_ref, wf_ref, bf_ref, wi_ref, bi_ref, wu_ref, bu_ref,
                wo_ref, bo_ref, h_ref, c_ref):
    s1 = s_ref[0, 0] + s_ref[0, 1]
    s2 = s_ref[1, 0] + s_ref[1, 1]
    s3 = s_ref[2, 0] + s_ref[2, 1]
    hsum = jnp.concatenate([s1, s2], axis=1)
    f = jax.nn.sigmoid(
        jnp.dot(hsum, wf_ref[...], preferred_element_type=jnp.float32)
        + bf_ref[...])
    i = jax.nn.sigmoid(
        jnp.dot(hsum, wi_ref[...], preferred_element_type=jnp.float32)
        + bi_ref[...])
    u = jnp.tanh(
        jnp.dot(hsum, wu_ref[...], preferred_element_type=jnp.float32)
        + bu_ref[...])
    o = jax.nn.sigmoid(
        jnp.dot(hsum, wo_ref[...], preferred_element_type=jnp.float32)
        + bo_ref[...])
    c = i * u + f * s3
    c_ref[...] = c
    h_ref[...] = o * jnp.tanh(c)


def _gates_stage(S, W_f, bias_f, W_i, bias_i, W_u, bias_u, W_o, bias_o):
    n_blk = N_NODES // NBLK
    wspec = pl.BlockSpec((2 * D, D), lambda i: (0, 0))
    bspec = pl.BlockSpec((1, D), lambda i: (0, 0))
    return pl.pallas_call(
        _gates_body,
        grid=(n_blk,),
        in_specs=[
            pl.BlockSpec((3, NC, NBLK, D), lambda i: (0, 0, i, 0)),
            wspec, bspec, wspec, bspec, wspec, bspec, wspec, bspec,
        ],
        out_specs=[
            pl.BlockSpec((NBLK, D), lambda i: (i, 0)),
            pl.BlockSpec((NBLK, D), lambda i: (i, 0)),
        ],
        out_shape=[
            jax.ShapeDtypeStruct((N_NODES, D), jnp.float32),
            jax.ShapeDtypeStruct((N_NODES, D), jnp.float32),
        ],
    )(S, W_f, bias_f, W_i, bias_i, W_u, bias_u, W_o, bias_o)


def kernel(h_child, c_child, embed_dst, src_node, dst_node, edge_type,
           dst_ids, W_f, bW_f, b_f, W_i, bW_i, b_i, W_u, bW_u, b_u, W_o,
           bW_o, b_o, W_eoh, bW_eoh, W_el, bW_el):
    zeros_hbm = jnp.zeros((NPAD, D), jnp.float32)
    msg = _msg_stage(h_child, src_node, dst_node, edge_type, W_eoh, bW_eoh,
                     W_el, bW_el)
    ids2d = dst_ids.reshape(NW, NCHD, CHUNK)
    S = _sc_segsum((msg, embed_dst, c_child), ids2d, zeros_hbm)
    h, c = _gates_stage(
        S,
        W_f, (bW_f + b_f).reshape(1, D),
        W_i, (bW_i + b_i).reshape(1, D),
        W_u, (bW_u + b_u).reshape(1, D),
        W_o, (bW_o + b_o).reshape(1, D),
    )
    return (h, c)


# speedup vs baseline: 1.0042x; 1.0042x over previous
import functools

import jax
import jax.numpy as jnp
from jax import lax
from jax.experimental import pallas as pl
from jax.experimental.pallas import tpu as pltpu
from jax.experimental.pallas import tpu_sc as plsc

N_NODES = 10000
N_EDGES = 320000
D = 128
NPAD = 10240

NC = 2
NS = 16
NW = NC * NS
EPW = N_EDGES // NW
CHUNK = 80
NCHD = EPW // CHUNK
RING = 3
PREF = 2
ROWS_PER_TILE = NPAD // NS

EBLK = 6400
NBLK = 1000


def _msg_body(et_ref, h_ref, s_ref, d_ref, weoh_ref, beoh_ref, wel_ref,
              bel_ref, out_ref):
    et = et_ref[0, 0, :][:, None]
    w0 = weoh_ref[0:1, :]
    w1 = weoh_ref[1:2, :]
    w2 = weoh_ref[2:3, :]
    etw = jnp.where(et == 0, w0, jnp.where(et == 1, w1, w2))
    a = s_ref[...] * d_ref[...] + etw + beoh_ref[...]
    ew = jnp.dot(a.astype(jnp.bfloat16), wel_ref[...].astype(jnp.bfloat16),
                 preferred_element_type=jnp.float32)
    out_ref[...] = h_ref[...] * (ew + bel_ref[...])


def _msg_stage(h_child, src_node, dst_node, edge_type, W_eoh, bW_eoh, W_el,
               bW_el):
    n_blk = N_EDGES // EBLK
    et3 = edge_type.reshape(n_blk, 1, EBLK)
    weoh_p = jnp.zeros((8, D), jnp.float32).at[:3].set(W_eoh)
    return pl.pallas_call(
        _msg_body,
        grid=(n_blk,),
        in_specs=[
            pl.BlockSpec((1, 1, EBLK), lambda i: (i, 0, 0)),
            pl.BlockSpec((EBLK, D), lambda i: (i, 0)),
            pl.BlockSpec((EBLK, D), lambda i: (i, 0)),
            pl.BlockSpec((EBLK, D), lambda i: (i, 0)),
            pl.BlockSpec((8, D), lambda i: (0, 0)),
            pl.BlockSpec((1, D), lambda i: (0, 0)),
            pl.BlockSpec((D, D), lambda i: (0, 0)),
            pl.BlockSpec((1, D), lambda i: (0, 0)),
        ],
        out_specs=pl.BlockSpec((EBLK, D), lambda i: (i, 0)),
        out_shape=jax.ShapeDtypeStruct((N_EDGES, D), jnp.float32),
    )(et3, h_child, src_node, dst_node, weoh_p, bW_eoh.reshape(1, D), W_el,
      bW_el.reshape(1, D))


def _sc_segsum(arrays, ids2d, zeros_hbm):
    n_arr = len(arrays)
    mesh = plsc.VectorSubcoreMesh(
        core_axis_name="c", subcore_axis_name="s", num_cores=NC,
        num_subcores=NS)

    slab_types = [pltpu.VMEM((CHUNK, D), jnp.float32)] * RING
    sem_types = [pltpu.SemaphoreType.DMA] * (2 * RING)

    @functools.partial(
        pl.kernel,
        out_type=jax.ShapeDtypeStruct((n_arr, NC, NPAD, D), jnp.float32),
        mesh=mesh,
        scratch_types=[
            pltpu.VMEM_SHARED((NPAD, D), jnp.float32),
            pltpu.VMEM((NCHD, CHUNK), jnp.int32),
        ] + slab_types + sem_types,
    )
    def k(*refs):
        val_hbms = refs[:n_arr]
        ids_hbm, zero_hbm, out_hbm, acc, ids_v = refs[n_arr:n_arr + 5]
        slab = refs[n_arr + 5:n_arr + 5 + RING]
        sem_ld = refs[n_arr + 5 + RING:n_arr + 5 + 2 * RING]
        sem_sc = refs[n_arr + 5 + 2 * RING:n_arr + 5 + 3 * RING]
        cid = lax.axis_index("c")
        sid = lax.axis_index("s")
        wid = sid * NC + cid
        base = wid * EPW
        row0 = sid * ROWS_PER_TILE

        pltpu.sync_copy(ids_hbm.at[wid], ids_v)

        def zero_acc():
            pltpu.sync_copy(zero_hbm.at[pl.ds(row0, ROWS_PER_TILE)],
                            acc.at[pl.ds(row0, ROWS_PER_TILE)])

        def load(vals, t, b):
            pltpu.async_copy(vals.at[pl.ds(base + t * CHUNK, CHUNK)],
                             slab[b], sem_ld[b])

        def wait_load(vals, b):
            pltpu.make_async_copy(vals.at[pl.ds(base, CHUNK)], slab[b],
                                  sem_ld[b]).wait()

        def fire_scatter(c, b):
            pltpu.async_copy(slab[b], acc.at[ids_v.at[c]], sem_sc[b],
                             add=True)

        def wait_scatter(b):
            pltpu.make_async_copy(slab[b], acc.at[ids_v.at[0]],
                                  sem_sc[b]).wait()

        zero_acc()
        plsc.subcore_barrier()

        for j, vals in enumerate(val_hbms):
            for i in range(PREF):
                load(vals, i, i)

            def grp(g, _, vals=vals):
                for i in range(RING):
                    c = RING * g + i
                    wait_load(vals, i)
                    fire_scatter(c, i)
                    t = c + PREF
                    bt = (i + PREF) % RING

                    def ws_ld(t=t, bt=bt):
                        wait_scatter(bt)
                        load(vals, t, bt)

                    if i == 0:
                        @pl.when(g > 0)
                        def _():
                            ws_ld()

                        @pl.when(g == 0)
                        def _(t=t, bt=bt):
                            load(vals, t, bt)
                    else:
                        ws_ld()
                return 0

            lax.fori_loop(0, NCHD // RING, grp, 0)
            for c in range(RING * (NCHD // RING), NCHD):
                b = c % RING
                wait_load(vals, b)
                fire_scatter(c, b)
            for b in range(RING):
                wait_scatter(b)
            plsc.subcore_barrier()
            pltpu.sync_copy(
                acc.at[pl.ds(row0, ROWS_PER_TILE)],
                out_hbm.at[j, cid, pl.ds(row0, ROWS_PER_TILE)])
            if j < n_arr - 1:
                zero_acc()
                plsc.subcore_barrier()

    return k(*arrays, ids2d, zeros_hbm)


def _gates_body(s_ref, wf_ref, bf_ref, wi_ref, bi_ref, wu_ref, bu_ref,
                wo_ref, bo_ref, h_ref, c_ref):
    s1 = s_ref[0, 0] + s_ref[0, 1]
    s2 = s_ref[1, 0] + s_ref[1, 1]
    s3 = s_ref[2, 0] + s_ref[2, 1]
    hsum = jnp.concatenate([s1, s2], axis=1)
    f = jax.nn.sigmoid(
        jnp.dot(hsum, wf_ref[...], preferred_element_type=jnp.float32)
        + bf_ref[...])
    i = jax.nn.sigmoid(
        jnp.dot(hsum, wi_ref[...], preferred_element_type=jnp.float32)
        + bi_ref[...])
    u = jnp.tanh(
        jnp.dot(hsum, wu_ref[...], preferred_element_type=jnp.float32)
        + bu_ref[...])
    o = jax.nn.sigmoid(
        jnp.dot(hsum, wo_ref[...], preferred_element_type=jnp.float32)
        + bo_ref[...])
    c = i * u + f * s3
    c_ref[...] = c
    h_ref[...] = o * jnp.tanh(c)


def _gates_stage(S, W_f, bias_f, W_i, bias_i, W_u, bias_u, W_o, bias_o):
    n_blk = N_NODES // NBLK
    wspec = pl.BlockSpec((2 * D, D), lambda i: (0, 0))
    bspec = pl.BlockSpec((1, D), lambda i: (0, 0))
    return pl.pallas_call(
        _gates_body,
        grid=(n_blk,),
        in_specs=[
            pl.BlockSpec((3, NC, NBLK, D), lambda i: (0, 0, i, 0)),
            wspec, bspec, wspec, bspec, wspec, bspec, wspec, bspec,
        ],
        out_specs=[
            pl.BlockSpec((NBLK, D), lambda i: (i, 0)),
            pl.BlockSpec((NBLK, D), lambda i: (i, 0)),
        ],
        out_shape=[
            jax.ShapeDtypeStruct((N_NODES, D), jnp.float32),
            jax.ShapeDtypeStruct((N_NODES, D), jnp.float32),
        ],
    )(S, W_f, bias_f, W_i, bias_i, W_u, bias_u, W_o, bias_o)


def kernel(h_child, c_child, embed_dst, src_node, dst_node, edge_type,
           dst_ids, W_f, bW_f, b_f, W_i, bW_i, b_i, W_u, bW_u, b_u, W_o,
           bW_o, b_o, W_eoh, bW_eoh, W_el, bW_el):
    zeros_hbm = jnp.zeros((NPAD, D), jnp.float32)
    msg = _msg_stage(h_child, src_node, dst_node, edge_type, W_eoh, bW_eoh,
                     W_el, bW_el)
    ids2d = dst_ids.reshape(NW, NCHD, CHUNK)
    S = _sc_segsum((msg, embed_dst, c_child), ids2d, zeros_hbm)
    h, c = _gates_stage(
        S,
        W_f, (bW_f + b_f).reshape(1, D),
        W_i, (bW_i + b_i).reshape(1, D),
        W_u, (bW_u + b_u).reshape(1, D),
        W_o, (bW_o + b_o).reshape(1, D),
    )
    return (h, c)
